# baseline (speedup 1.0000x reference)
import functools

import jax
import jax.numpy as jnp
import numpy as np
from jax import lax
from jax.experimental import pallas as pl
from jax.experimental.pallas import tpu as pltpu
from jax.experimental.pallas import tpu_sc as plsc

_C = 100000
_B = 1024
_SMOOTH = 0.1

_FILL = float(np.float32(_SMOOTH / (_C - 1)))
_ON = float(np.float32(1.0 - _SMOOTH))
_CONST = _B * ((_C - 1) * _FILL * np.log(_FILL) + _ON * np.log(_ON))
_INV_N = 1.0 / (_B * _C)
_K0 = np.float32(_CONST * _INV_N)
_K1 = np.float32(-_FILL * _INV_N)
_K2 = np.float32(-(_ON - _FILL) * _INV_N)

_NC, _NS, _NL = 2, 16, 16
_NW = _NC * _NS
_RPW = _B // _NW

_C_MAIN = 99968
_C_TAIL = _C - _C_MAIN

_SC_ROWS = 512
_SC_BPW = _SC_ROWS // (8 * _NW)
_CW = 8192
_NCH = _C_MAIN // _CW
_TW = _C_MAIN - _NCH * _CW

_TC_BLK_ROWS = 32
_TC_GRID = (_B - _SC_ROWS) // _TC_BLK_ROWS


def _row_sums(buf, width, acc, unroll=8):
    step = _NL * unroll
    for r in range(8):
        def inner(j, a, r=r):
            s = a
            for u in range(unroll):
                s = s + buf[r, pl.ds(j * step + u * _NL, _NL)]
            return s
        acc = lax.fori_loop(0, width // step, inner, acc)
    return acc


def _sc_body(x, tgt, out, t_v, tiles_v, part_v, buf_v, tail_v, sem):
    wid = lax.axis_index("s") * _NC + lax.axis_index("c")
    base = wid * _RPW
    pltpu.sync_copy(tgt.at[pl.ds(base, _RPW)], t_v)
    copies = []
    for h in range(_RPW // _NL):
        tcb = t_v[pl.ds(h * _NL, _NL)] & -128
        for l in range(_NL):
            j = h * _NL + l
            rowb = pl.multiple_of((base + j) & -8, 8)
            colb = pl.multiple_of(tcb[l], 128)
            copies.append(
                pltpu.async_copy(
                    x.at[pl.ds(rowb, 8), pl.ds(colb, 128)],
                    tiles_v.at[j],
                    sem,
                )
            )
    for c in copies:
        c.wait()
    acc = None
    lane = lax.iota(jnp.int32, _NL)
    for h in range(_RPW // _NL):
        tv = t_v[pl.ds(h * _NL, _NL)]
        tgl = tv & (128 - _NL)
        tcol = tv & (_NL - 1)
        for l in range(_NL):
            j = h * _NL + l
            row16 = tiles_v[j, (base + j) % 8, pl.ds(tgl[l], _NL)]
            sel = jnp.where(lane == tcol[l], row16, 0.0)
            acc = sel if acc is None else acc + sel
    part_v[...] = acc
    pltpu.sync_copy(part_v, out.at[0, wid])
    acc2 = jnp.zeros((_NL,), jnp.float32)
    for k in range(_SC_BPW):
        row8 = pl.multiple_of((k * _NW + wid) * 8, 8)

        def chunk(c, a, row8=row8):
            col = pl.multiple_of(c * _CW, 128)
            pltpu.sync_copy(x.at[pl.ds(row8, 8), pl.ds(col, _CW)], buf_v)
            return _row_sums(buf_v, _CW, a)

        acc2 = lax.fori_loop(0, _NCH, chunk, acc2)
        col = pl.multiple_of(_NCH * _CW, 128)
        pltpu.sync_copy(x.at[pl.ds(row8, 8), pl.ds(col, _TW)], tail_v)
        acc2 = _row_sums(tail_v, _TW, acc2)
    part_v[...] = acc2
    pltpu.sync_copy(part_v, out.at[1, wid])


@functools.cache
def _sc_call():
    return functools.partial(
        pl.kernel,
        mesh=plsc.VectorSubcoreMesh(core_axis_name="c", subcore_axis_name="s"),
        out_type=jax.ShapeDtypeStruct((2, _NW, _NL), jnp.float32),
        scratch_types=[
            pltpu.VMEM((_RPW,), jnp.int32),
            pltpu.VMEM((_RPW, 8, 128), jnp.float32),
            pltpu.VMEM((_NL,), jnp.float32),
            pltpu.VMEM((8, _CW), jnp.float32),
            pltpu.VMEM((8, _TW), jnp.float32),
            pltpu.SemaphoreType.DMA,
        ],
    )(_sc_body)


def _tc_sum_body(x_ref, sl_ref, out_ref, acc_ref):
    i = pl.program_id(0)

    @pl.when(i == 0)
    def _init():
        acc_ref[0, 0] = 0.0

    acc_ref[0, 0] += jnp.sum(x_ref[...])

    @pl.when(i == _TC_GRID - 1)
    def _fin():
        lane = lax.broadcasted_iota(jnp.int32, (_B, 128), 1)
        sl = jnp.sum(jnp.where(lane < _C_TAIL, sl_ref[...], 0.0))
        out_ref[0, 0] = acc_ref[0, 0] + sl


def _combine_body(s_ref, p_ref, out_ref):
    g = jnp.sum(p_ref[0])
    ssc = jnp.sum(p_ref[1])
    out_ref[0, 0] = _K0 + _K1 * (s_ref[0, 0] + ssc) + _K2 * g


def kernel(X, target):
    sc_parts = _sc_call()(X, target)
    tc_sum = pl.pallas_call(
        _tc_sum_body,
        grid=(_TC_GRID,),
        in_specs=[
            pl.BlockSpec(
                (_TC_BLK_ROWS, _C_MAIN),
                lambda i: (i + _SC_ROWS // _TC_BLK_ROWS, 0),
            ),
            pl.BlockSpec((_B, 128), lambda i: (0, _C_MAIN // 128)),
        ],
        out_specs=pl.BlockSpec(
            (1, 1), lambda i: (0, 0), memory_space=pltpu.SMEM
        ),
        out_shape=jax.ShapeDtypeStruct((1, 1), jnp.float32),
        scratch_shapes=[pltpu.SMEM((1, 1), jnp.float32)],
    )(X, X)
    out = pl.pallas_call(
        _combine_body,
        in_specs=[
            pl.BlockSpec(memory_space=pltpu.SMEM),
            pl.BlockSpec((2, _NW, _NL), lambda: (0, 0, 0)),
        ],
        out_specs=pl.BlockSpec(memory_space=pltpu.SMEM),
        out_shape=jax.ShapeDtypeStruct((1, 1), jnp.float32),
    )(tc_sum, sc_parts)
    return out.reshape(())


# speedup vs baseline: 3.8808x; 3.8808x over previous
import functools

import jax
import jax.numpy as jnp
import numpy as np
from jax import lax
from jax.experimental import pallas as pl
from jax.experimental.pallas import tpu as pltpu
from jax.experimental.pallas import tpu_sc as plsc

_C = 100000
_B = 1024
_SMOOTH = 0.1

_FILL = float(np.float32(_SMOOTH / (_C - 1)))
_ON = float(np.float32(1.0 - _SMOOTH))
_CONST = _B * ((_C - 1) * _FILL * np.log(_FILL) + _ON * np.log(_ON))
_INV_N = 1.0 / (_B * _C)
_K0 = np.float32(_CONST * _INV_N)
_K1 = np.float32(-_FILL * _INV_N)
_K2 = np.float32(-(_ON - _FILL) * _INV_N)

_NC, _NS, _NL = 2, 16, 16
_NW = _NC * _NS
_RPW = _B // _NW

_TC_BLK = 4000
_TC_GRID = _C // _TC_BLK


def _sc_body(xt, tgt, out, t_v, tiles_v, part_v, sem):
    wid = lax.axis_index("s") * _NC + lax.axis_index("c")
    base = wid * _RPW
    pltpu.sync_copy(tgt.at[pl.ds(base, _RPW)], t_v)
    copies = []
    for h in range(_RPW // _NL):
        tv8 = t_v[pl.ds(h * _NL, _NL)] & -8
        for l in range(_NL):
            j = h * _NL + l
            rowb = pl.multiple_of(tv8[l], 8)
            colb = ((base + j) // 128) * 128
            copies.append(
                pltpu.async_copy(
                    xt.at[pl.ds(rowb, 8), pl.ds(colb, 128)],
                    tiles_v.at[j],
                    sem,
                )
            )
    for c in copies:
        c.wait()
    acc = None
    lane = lax.iota(jnp.int32, _NL)
    for h in range(_RPW // _NL):
        tv = t_v[pl.ds(h * _NL, _NL)]
        trow = tv & 7
        for l in range(_NL):
            j = h * _NL + l
            lb = ((base + j) % 128) & -_NL
            row16 = tiles_v[j, trow[l], pl.ds(lb, _NL)]
            sel = jnp.where(lane == (base + j) % _NL, row16, 0.0)
            acc = sel if acc is None else acc + sel
    part_v[...] = acc
    pltpu.sync_copy(part_v, out.at[wid])


@functools.cache
def _sc_call():
    return functools.partial(
        pl.kernel,
        mesh=plsc.VectorSubcoreMesh(core_axis_name="c", subcore_axis_name="s"),
        out_type=jax.ShapeDtypeStruct((_NW, _NL), jnp.float32),
        scratch_types=[
            pltpu.VMEM((_RPW,), jnp.int32),
            pltpu.VMEM((_RPW, 8, 128), jnp.float32),
            pltpu.VMEM((_NL,), jnp.float32),
            pltpu.SemaphoreType.DMA,
        ],
    )(_sc_body)


def _tc_sum_body(x_ref, out_ref, acc_ref):
    i = pl.program_id(0)

    @pl.when(i == 0)
    def _init():
        acc_ref[0, 0] = 0.0

    acc_ref[0, 0] += jnp.sum(x_ref[...])

    @pl.when(i == _TC_GRID - 1)
    def _fin():
        out_ref[0, 0] = acc_ref[0, 0]


def _combine_body(s_ref, p_ref, out_ref):
    g = jnp.sum(p_ref[...])
    out_ref[0, 0] = _K0 + _K1 * s_ref[0, 0] + _K2 * g


def kernel(X, target):
    xt = X.T
    sc_parts = _sc_call()(xt, target)
    tc_sum = pl.pallas_call(
        _tc_sum_body,
        grid=(_TC_GRID,),
        in_specs=[pl.BlockSpec((_TC_BLK, _B), lambda i: (i, 0))],
        out_specs=pl.BlockSpec(
            (1, 1), lambda i: (0, 0), memory_space=pltpu.SMEM
        ),
        out_shape=jax.ShapeDtypeStruct((1, 1), jnp.float32),
        scratch_shapes=[pltpu.SMEM((1, 1), jnp.float32)],
    )(xt)
    out = pl.pallas_call(
        _combine_body,
        in_specs=[
            pl.BlockSpec(memory_space=pltpu.SMEM),
            pl.BlockSpec((_NW, _NL), lambda: (0, 0)),
        ],
        out_specs=pl.BlockSpec(memory_space=pltpu.SMEM),
        out_shape=jax.ShapeDtypeStruct((1, 1), jnp.float32),
    )(tc_sum, sc_parts)
    return out.reshape(())
